# baseline (speedup 1.0000x reference)
import functools

import jax
import jax.numpy as jnp
from jax import lax
from jax.experimental import pallas as pl
from jax.experimental.pallas import tpu as pltpu
from jax.experimental.pallas import tpu_sc as plsc

EMB_DIM = 64
NUM_CORES = 2
NUM_SUBCORES = 16
NUM_WORKERS = NUM_CORES * NUM_SUBCORES
CHUNK = 800
NBUF = 2


@functools.partial(jax.jit, static_argnums=(2,))
def _gather_rows(ids, table, batch):
    per_worker = batch // NUM_WORKERS
    n_chunks = per_worker // CHUNK
    assert per_worker % CHUNK == 0 and n_chunks % NBUF == 0
    mesh = plsc.VectorSubcoreMesh(
        core_axis_name="c", subcore_axis_name="s",
        num_cores=NUM_CORES, num_subcores=NUM_SUBCORES)

    @functools.partial(
        pl.kernel,
        mesh=mesh,
        compiler_params=pltpu.CompilerParams(use_tc_tiling_on_sc=False),
        out_type=jax.ShapeDtypeStruct((batch, EMB_DIM), jnp.float32),
        scratch_types=[
            pltpu.VMEM((NBUF, CHUNK), jnp.int32),
            pltpu.VMEM((NBUF, CHUNK, EMB_DIM), jnp.float32),
            pltpu.SemaphoreType.DMA((NBUF,)),
            pltpu.SemaphoreType.DMA((NBUF,)),
            pltpu.SemaphoreType.DMA((NBUF,)),
        ],
    )
    def body(ids_hbm, table_hbm, out_hbm, idx_v, rows_v, sem_i, sem_g, sem_o):
        wid = lax.axis_index("s") * NUM_CORES + lax.axis_index("c")
        base = wid * per_worker

        def ids_slice(i):
            return ids_hbm.at[pl.ds(base + i * CHUNK, CHUNK)]

        def out_slice(i):
            return out_hbm.at[pl.ds(base + i * CHUNK, CHUNK)]

        for b in range(NBUF):
            pltpu.async_copy(ids_slice(b), idx_v.at[b], sem_i.at[b])

        @pl.loop(0, n_chunks, step=NBUF)
        def _(i):
            for b in range(NBUF):
                ib = i + b

                @pl.when(ib >= NBUF)
                def _():
                    pltpu.make_async_copy(
                        rows_v.at[b], out_slice(ib - NBUF), sem_o.at[b]).wait()

                pltpu.make_async_copy(
                    ids_slice(ib), idx_v.at[b], sem_i.at[b]).wait()

                pltpu.async_copy(
                    table_hbm.at[idx_v.at[b]], rows_v.at[b], sem_g.at[b]).wait()

                @pl.when(ib + NBUF < n_chunks)
                def _():
                    pltpu.async_copy(
                        ids_slice(ib + NBUF), idx_v.at[b], sem_i.at[b])

                pltpu.async_copy(rows_v.at[b], out_slice(ib), sem_o.at[b])

        for b in range(NBUF):
            pltpu.make_async_copy(
                rows_v.at[b], out_slice(n_chunks - NBUF + b), sem_o.at[b]).wait()

    return body(ids, table)


def kernel(input, table):
    n, l = input.shape
    ids = (input.astype(jnp.int32) * 2).T.reshape(-1)
    tbl2 = jnp.pad(table, ((0, 0), (0, 64))).reshape(2 * table.shape[0], 64)
    out = _gather_rows(ids, tbl2, n * l)
    return out.reshape(l, n, EMB_DIM).transpose(1, 0, 2)


# speedup vs baseline: 1.0016x; 1.0016x over previous
import functools

import jax
import jax.numpy as jnp
from jax import lax
from jax.experimental import pallas as pl
from jax.experimental.pallas import tpu as pltpu
from jax.experimental.pallas import tpu_sc as plsc

EMB_DIM = 64
NUM_CORES = 2
NUM_SUBCORES = 16
NUM_WORKERS = NUM_CORES * NUM_SUBCORES
CHUNK = 640
NBUF = 2


@functools.partial(jax.jit, static_argnums=(2,))
def _gather_rows(ids, table, batch):
    per_worker = batch // NUM_WORKERS
    n_chunks = per_worker // CHUNK
    assert per_worker % CHUNK == 0 and n_chunks % NBUF == 0
    mesh = plsc.VectorSubcoreMesh(
        core_axis_name="c", subcore_axis_name="s",
        num_cores=NUM_CORES, num_subcores=NUM_SUBCORES)

    @functools.partial(
        pl.kernel,
        mesh=mesh,
        compiler_params=pltpu.CompilerParams(use_tc_tiling_on_sc=False),
        out_type=jax.ShapeDtypeStruct((batch, EMB_DIM), jnp.float32),
        scratch_types=[
            pltpu.VMEM((NBUF, CHUNK), jnp.int32),
            pltpu.VMEM((NBUF, CHUNK, EMB_DIM), jnp.float32),
            pltpu.SemaphoreType.DMA((NBUF,)),
            pltpu.SemaphoreType.DMA((NBUF,)),
            pltpu.SemaphoreType.DMA((NBUF,)),
        ],
    )
    def body(ids_hbm, table_hbm, out_hbm, idx_v, rows_v, sem_i, sem_g, sem_o):
        wid = lax.axis_index("s") * NUM_CORES + lax.axis_index("c")
        base = wid * per_worker

        def ids_slice(i):
            return ids_hbm.at[pl.ds(base + i * CHUNK, CHUNK)]

        def out_slice(i):
            return out_hbm.at[pl.ds(base + i * CHUNK, CHUNK)]

        for b in range(NBUF):
            pltpu.async_copy(ids_slice(b), idx_v.at[b], sem_i.at[b])

        @pl.loop(0, n_chunks, step=NBUF)
        def _(i):
            for b in range(NBUF):
                ib = i + b

                @pl.when(ib >= NBUF)
                def _():
                    pltpu.make_async_copy(
                        rows_v.at[b], out_slice(ib - NBUF), sem_o.at[b]).wait()

                pltpu.make_async_copy(
                    ids_slice(ib), idx_v.at[b], sem_i.at[b]).wait()

                pltpu.async_copy(
                    table_hbm.at[idx_v.at[b]], rows_v.at[b], sem_g.at[b]).wait()

                @pl.when(ib + NBUF < n_chunks)
                def _():
                    pltpu.async_copy(
                        ids_slice(ib + NBUF), idx_v.at[b], sem_i.at[b])

                pltpu.async_copy(rows_v.at[b], out_slice(ib), sem_o.at[b])

        for b in range(NBUF):
            pltpu.make_async_copy(
                rows_v.at[b], out_slice(n_chunks - NBUF + b), sem_o.at[b]).wait()

    return body(ids, table)


def kernel(input, table):
    n, l = input.shape
    ids = (input.astype(jnp.int32) * 2).T.reshape(-1)
    tbl2 = jnp.pad(table, ((0, 0), (0, 64))).reshape(2 * table.shape[0], 64)
    out = _gather_rows(ids, tbl2, n * l)
    return out.reshape(l, n, EMB_DIM).transpose(1, 0, 2)
